# baseline (speedup 1.0000x reference)
import functools

import jax
import jax.numpy as jnp
from jax import lax
from jax.experimental import pallas as pl
from jax.experimental.pallas import tpu as pltpu
from jax.experimental.pallas import tpu_sc as plsc

MIN_NORM = 1e-15
CLIP = 1e-7


def _artanh(z):
    return 0.5 * jnp.log((1.0 + z) / (1.0 - z))

def _k1_body(x_ref, w1a_ref, w1b_ref, a_ref, b_ref):
    xb = x_ref[...]
    t = jnp.sum(xb * xb, axis=1, keepdims=True)
    pn = jnp.maximum(jnp.sqrt(t), MIN_NORM)
    z = jnp.clip(pn, -1 + CLIP, 1 - CLIP)
    s = _artanh(z) / pn
    xt = xb * s
    a_ref[...] = jnp.dot(
        xt, w1a_ref[...], preferred_element_type=jnp.float32
    ).astype(jnp.bfloat16)
    b_ref[...] = jnp.dot(
        xt, w1b_ref[...], preferred_element_type=jnp.float32
    ).astype(jnp.bfloat16)


def _node_pre(x, w1a, w1b, blk=1000):
    n, d = x.shape
    grid = n // blk
    return pl.pallas_call(
        _k1_body,
        grid=(grid,),
        in_specs=[
            pl.BlockSpec((blk, d), lambda i: (i, 0)),
            pl.BlockSpec((d, d), lambda i: (0, 0)),
            pl.BlockSpec((d, d), lambda i: (0, 0)),
        ],
        out_specs=[
            pl.BlockSpec((blk, d), lambda i: (i, 0)),
            pl.BlockSpec((blk, d), lambda i: (i, 0)),
        ],
        out_shape=[
            jax.ShapeDtypeStruct((n, d), jnp.bfloat16),
            jax.ShapeDtypeStruct((n, d), jnp.bfloat16),
        ],
    )(x, w1a, w1b)


def _sc_gather(x, a_tab, b_tab, idxr2, idxc2, chunk, n_chunks_per_w):
    n, d = x.shape
    nrows, ck = idxr2.shape
    e = nrows * ck
    nch = n_chunks_per_w
    mesh = plsc.VectorSubcoreMesh(core_axis_name="c", subcore_axis_name="s")
    nc, ns = mesh.num_cores, mesh.num_subcores
    nw = nc * ns
    e_per_w = e // nw

    slot_bufs = []
    for _ in range(3):
        slot_bufs += [
            pltpu.VMEM((ck,), jnp.int32),
            pltpu.VMEM((ck,), jnp.int32),
            pltpu.VMEM((ck, d), jnp.float32),
            pltpu.VMEM((ck, d), jnp.float32),
            pltpu.VMEM((ck, d // 2), jnp.float32),
            pltpu.VMEM((ck, d // 2), jnp.float32),
        ]
    sems = [pltpu.SemaphoreType.DMA] * 9

    @functools.partial(
        pl.kernel,
        out_type=[
            jax.ShapeDtypeStruct((e, d), jnp.float32),
            jax.ShapeDtypeStruct((e, d), jnp.float32),
            jax.ShapeDtypeStruct((e, d // 2), jnp.float32),
            jax.ShapeDtypeStruct((e, d // 2), jnp.float32),
        ],
        mesh=mesh,
        scratch_types=slot_bufs + sems,
        compiler_params=pltpu.CompilerParams(use_tc_tiling_on_sc=False),
    )
    def k2(x_hbm, a_hbm, b_hbm, idxr_hbm, idxc_hbm,
           xr_out, xc_out, ap_out, bp_out, *scr):
        bufs = [scr[6 * s:6 * s + 6] for s in range(3)]
        semi = scr[18:21]
        semg = scr[21:24]
        semw = scr[24:27]
        cid = lax.axis_index("c")
        sid = lax.axis_index("s")
        wid = sid * nc + cid

        def issue_idx(s, j):
            crow = wid * nch + j
            pltpu.async_copy(idxr_hbm.at[crow], bufs[s][0], semi[s])
            pltpu.async_copy(idxc_hbm.at[crow], bufs[s][1], semi[s])

        def wait_idx(s):
            pltpu.make_async_copy(idxr_hbm.at[0], bufs[s][0], semi[s]).wait()
            pltpu.make_async_copy(idxr_hbm.at[0], bufs[s][1], semi[s]).wait()

        def issue_gathers(s):
            ir, ic, xr_v, xc_v, aq_v, bc_v = bufs[s]
            pltpu.async_copy(x_hbm.at[ir], xr_v, semg[s])
            pltpu.async_copy(x_hbm.at[ic], xc_v, semg[s])
            pltpu.async_copy(a_hbm.at[ir], aq_v, semg[s])
            pltpu.async_copy(b_hbm.at[ic], bc_v, semg[s])

        def wait_gathers(s):
            for b in (2, 3):
                pltpu.make_async_copy(
                    xr_out.at[pl.ds(0, ck)], bufs[s][b], semg[s]).wait()
            for b in (4, 5):
                pltpu.make_async_copy(
                    ap_out.at[pl.ds(0, ck)], bufs[s][b], semg[s]).wait()

        def compute_q(s):
            pass

        def issue_writes(s, j):
            gbase = wid * e_per_w + j * chunk
            pltpu.async_copy(bufs[s][2], xr_out.at[pl.ds(gbase, ck)], semw[s])
            pltpu.async_copy(bufs[s][3], xc_out.at[pl.ds(gbase, ck)], semw[s])
            pltpu.async_copy(bufs[s][4], ap_out.at[pl.ds(gbase, ck)], semw[s])
            pltpu.async_copy(bufs[s][5], bp_out.at[pl.ds(gbase, ck)], semw[s])

        def wait_writes(s):
            for b in (2, 3):
                pltpu.make_async_copy(
                    bufs[s][b], xr_out.at[pl.ds(0, ck)], semw[s]).wait()
            for b in (4, 5):
                pltpu.make_async_copy(
                    bufs[s][b], ap_out.at[pl.ds(0, ck)], semw[s]).wait()

        def step_full(j, sj):
            s2 = (sj + 1) % 3
            s3 = (sj + 2) % 3
            wait_gathers(sj)
            compute_q(sj)
            issue_writes(sj, j)
            wait_idx(s2)
            issue_gathers(s2)
            wait_writes(s3)
            issue_idx(s3, j + 2)

        issue_idx(0, 0)
        wait_idx(0)
        issue_gathers(0)
        issue_idx(1, 1)
        wait_gathers(0)
        compute_q(0)
        issue_writes(0, 0)
        wait_idx(1)
        issue_gathers(1)
        issue_idx(2, 2)

        assert nch >= 5
        nit = (nch - 5) // 3

        def body(jj, carry):
            j0 = 1 + 3 * jj
            step_full(j0, 1)
            step_full(j0 + 1, 2)
            step_full(j0 + 2, 0)
            return carry

        lax.fori_loop(0, nit, body, 0)
        for j in range(1 + 3 * nit, nch - 2):
            step_full(j, j % 3)
        j = nch - 2
        sj = j % 3
        wait_gathers(sj)
        compute_q(sj)
        issue_writes(sj, j)
        wait_idx((sj + 1) % 3)
        issue_gathers((sj + 1) % 3)
        j = nch - 1
        sj = j % 3
        wait_gathers(sj)
        compute_q(sj)
        issue_writes(sj, j)
        for s in range(3):
            wait_writes(s)

    return k2(x, a_tab, b_tab, idxr2, idxc2)


def _k3_body(xr_ref, xc_ref, ap_ref, bp_ref, d_ref, em_ref,
             w1ce_ref, w1co_ref, b1e_ref, b1o_ref, w2e_ref, w2o_ref,
             b2_ref, y_ref):
    xr = xr_ref[...]
    xc = xc_ref[...]
    ai = lax.bitcast_convert_type(ap_ref[...], jnp.int32)
    bi = lax.bitcast_convert_type(bp_ref[...], jnp.int32)
    mask_hi = jnp.int32(-65536)
    a_e = lax.bitcast_convert_type(ai << 16, jnp.float32)
    a_o = lax.bitcast_convert_type(ai & mask_hi, jnp.float32)
    b_e = lax.bitcast_convert_type(bi << 16, jnp.float32)
    b_o = lax.bitcast_convert_type(bi & mask_hi, jnp.float32)
    dist = d_ref[...]
    qe = a_e + b_e + dist * w1ce_ref[...] + b1e_ref[...]
    qo = a_o + b_o + dist * w1co_ref[...] + b1o_ref[...]
    he = qe * jax.nn.sigmoid(qe)
    ho = qo * jax.nn.sigmoid(qo)
    att = jax.nn.sigmoid(
        jnp.dot(he, w2e_ref[...], preferred_element_type=jnp.float32)
        + jnp.dot(ho, w2o_ref[...], preferred_element_type=jnp.float32)
        + b2_ref[...]) * em_ref[...]
    tr = jnp.sum(xr * xr, axis=1, keepdims=True)
    tc = jnp.sum(xc * xc, axis=1, keepdims=True)
    xy = jnp.sum(xr * xc, axis=1, keepdims=True)
    c1 = 1.0 + 2.0 * (-xy) + tc
    c2 = 1.0 - tr
    den = jnp.maximum(1.0 + 2.0 * (-xy) + tr * tc, MIN_NORM)
    sub = (c1 * (-xr) + c2 * xc) / den
    u = jnp.sum(sub * sub, axis=1, keepdims=True)
    sn = jnp.maximum(jnp.sqrt(u), MIN_NORM)
    z = jnp.clip(sn, -1 + CLIP, 1 - CLIP)
    fac = jnp.maximum(1.0 - tr, MIN_NORM) * _artanh(z) / sn
    y_ref[...] = (att * fac) * sub


def _edge_dense(xr, xc, ap, bp, dist, em, w1ce, w1co, b1e, b1o, w2e, w2o,
                b2r, blk=2560):
    e, d = xr.shape
    dh = d // 2
    grid = e // blk
    em_spec = pl.BlockSpec((blk, 1), lambda i: (i, 0))
    row_spec = pl.BlockSpec((blk, d), lambda i: (i, 0))
    half_spec = pl.BlockSpec((blk, dh), lambda i: (i, 0))
    vh_spec = pl.BlockSpec((1, dh), lambda i: (0, 0))
    return pl.pallas_call(
        _k3_body,
        grid=(grid,),
        in_specs=[
            row_spec, row_spec, half_spec, half_spec,
            em_spec, em_spec,
            vh_spec, vh_spec, vh_spec, vh_spec,
            pl.BlockSpec((dh, 1), lambda i: (0, 0)),
            pl.BlockSpec((dh, 1), lambda i: (0, 0)),
            pl.BlockSpec((1, 1), lambda i: (0, 0)),
        ],
        out_specs=row_spec,
        out_shape=jax.ShapeDtypeStruct((e, d), jnp.float32),
    )(xr, xc, ap, bp, dist, em, w1ce, w1co, b1e, b1o, w2e, w2o, b2r)


def _sc_scatter(y, idxr2, zeros_nd, n, chunk, n_chunks_per_w):
    e, d = y.shape
    nch = n_chunks_per_w
    mesh = plsc.VectorSubcoreMesh(core_axis_name="c", subcore_axis_name="s")
    nc, ns = mesh.num_cores, mesh.num_subcores
    nw = nc * ns
    e_per_w = e // nw
    bs = (n // ns) & ~7
    tail = n - ns * bs

    slot_bufs = []
    for _ in range(3):
        slot_bufs += [
            pltpu.VMEM((chunk,), jnp.int32),
            pltpu.VMEM((chunk, d), jnp.float32),
        ]

    @functools.partial(
        pl.kernel,
        out_type=jax.ShapeDtypeStruct((nc, n, d), jnp.float32),
        mesh=mesh,
        scratch_types=slot_bufs + [pltpu.VMEM_SHARED((n, d), jnp.float32)]
        + [pltpu.SemaphoreType.DMA] * 6,
    )
    def k4(y_hbm, idxr_hbm, z_hbm, out_hbm, *scr):
        bufs = [scr[2 * s:2 * s + 2] for s in range(3)]
        agg_sh = scr[6]
        seml = scr[7:10]
        semsc = scr[10:13]
        cid = lax.axis_index("c")
        sid = lax.axis_index("s")
        wid = sid * nc + cid
        myrows = pl.ds(sid * bs, bs)
        tailrows = pl.ds(ns * bs, tail)
        pltpu.sync_copy(z_hbm.at[myrows], agg_sh.at[myrows])
        if tail:
            @pl.when(sid == ns - 1)
            def _():
                pltpu.sync_copy(z_hbm.at[tailrows], agg_sh.at[tailrows])
        plsc.subcore_barrier()

        def issue_loads(s, j):
            crow = wid * nch + j
            gbase = wid * e_per_w + j * chunk
            pltpu.async_copy(idxr_hbm.at[crow], bufs[s][0], seml[s])
            pltpu.async_copy(y_hbm.at[pl.ds(gbase, chunk)], bufs[s][1],
                             seml[s])

        def wait_loads(s):
            pltpu.make_async_copy(idxr_hbm.at[0], bufs[s][0], seml[s]).wait()
            pltpu.make_async_copy(y_hbm.at[pl.ds(0, chunk)], bufs[s][1],
                                  seml[s]).wait()

        def issue_scatter(s):
            pltpu.async_copy(bufs[s][1], agg_sh.at[bufs[s][0]], semsc[s],
                             add=True)

        def wait_scatter(s):
            pltpu.make_async_copy(bufs[s][1], agg_sh.at[bufs[s][0]],
                                  semsc[s]).wait()

        def step_full(j, sj):
            s3 = (sj + 2) % 3
            wait_loads(sj)
            issue_scatter(sj)
            wait_scatter(s3)
            issue_loads(s3, j + 2)

        issue_loads(0, 0)
        issue_loads(1, 1)
        wait_loads(0)
        issue_scatter(0)
        issue_loads(2, 2)

        assert nch >= 5
        nit = (nch - 5) // 3

        def body(jj, carry):
            j0 = 1 + 3 * jj
            step_full(j0, 1)
            step_full(j0 + 1, 2)
            step_full(j0 + 2, 0)
            return carry

        lax.fori_loop(0, nit, body, 0)
        for j in range(1 + 3 * nit, nch - 2):
            step_full(j, j % 3)
        j = nch - 2
        wait_loads(j % 3)
        issue_scatter(j % 3)
        j = nch - 1
        wait_loads(j % 3)
        issue_scatter(j % 3)
        for s in range(3):
            wait_scatter(s)
        plsc.subcore_barrier()
        pltpu.sync_copy(agg_sh.at[myrows], out_hbm.at[cid, myrows])
        if tail:
            @pl.when(sid == ns - 1)
            def _():
                pltpu.sync_copy(agg_sh.at[tailrows], out_hbm.at[cid, tailrows])

    return k4(y, idxr2, zeros_nd)


def _k5_body(a0_ref, a1_ref, x_ref, wn1_ref, bn1_ref, wn2_ref, bn2_ref,
             out_ref):
    agg = (a0_ref[...] + a1_ref[...]) * 0.01
    m1 = jnp.dot(agg, wn1_ref[...], preferred_element_type=jnp.float32) \
        + bn1_ref[...]
    m1 = m1 * jax.nn.sigmoid(m1)
    m = jnp.dot(m1, wn2_ref[...], preferred_element_type=jnp.float32) \
        + bn2_ref[...]
    xb = x_ref[...]
    t = jnp.sum(xb * xb, axis=1, keepdims=True)
    n1 = 1.0 + 2.0 * (-t) + t
    n2 = 1.0 - t
    num = n1 * (-xb) + n2 * xb
    den = jnp.maximum(1.0 + 2.0 * (-t) + t * t, MIN_NORM)
    sb = num / den
    sn = jnp.maximum(jnp.sqrt(jnp.sum(sb * sb, axis=1, keepdims=True)),
                     MIN_NORM)
    lam = 2.0 / jnp.maximum(1.0 - t, MIN_NORM)
    zc = jnp.clip(sn, -1 + CLIP, 1 - CLIP)
    xlst = (2.0 / lam) * _artanh(zc) * sb / sn
    sup = xlst + m
    un = jnp.maximum(jnp.sqrt(jnp.sum(sup * sup, axis=1, keepdims=True)),
                     MIN_NORM)
    second = jnp.tanh(lam * un / 2.0) * sup / un
    y2 = jnp.sum(second * second, axis=1, keepdims=True)
    xy = jnp.sum(xb * second, axis=1, keepdims=True)
    numo = (1.0 + 2.0 * xy + y2) * xb + (1.0 - t) * second
    deno = jnp.maximum(1.0 + 2.0 * xy + t * y2, MIN_NORM)
    out_ref[...] = numo / deno


def _node_post(a0, a1, x, wn1, bn1, wn2, bn2, blk=1000):
    n, d = x.shape
    grid = n // blk
    row_spec = pl.BlockSpec((blk, d), lambda i: (i, 0))
    w_spec = pl.BlockSpec((d, d), lambda i: (0, 0))
    b_spec = pl.BlockSpec((1, d), lambda i: (0, 0))
    return pl.pallas_call(
        _k5_body,
        grid=(grid,),
        in_specs=[row_spec, row_spec, row_spec, w_spec, b_spec, w_spec,
                  b_spec],
        out_specs=row_spec,
        out_shape=jax.ShapeDtypeStruct((n, d), jnp.float32),
    )(a0, a1, x, wn1, bn1, wn2, bn2)


def kernel(x, distances, edges, node_mask, edge_mask, W1, b1, W2, b2,
           Wn1, bn1, Wn2, bn2):
    n, d = x.shape
    e = distances.shape[0]
    chunk = 80
    nw = 32
    e_per_w = e // nw
    n_chunks_per_w = e_per_w // chunk

    row = edges[0].astype(jnp.int32)
    col = edges[1].astype(jnp.int32)
    idxr2 = row.reshape(e // chunk, chunk)
    idxc2 = col.reshape(e // chunk, chunk)

    w1a = W1[:d]
    w1b = W1[d:2 * d]
    w1c = W1[2 * d:].reshape(1, d)
    b1r = b1.reshape(1, d)
    b2r = b2.reshape(1, 1)
    bn1r = bn1.reshape(1, d)
    bn2r = bn2.reshape(1, d)

    a_tab, b_tab = _node_pre(x, w1a, w1b)
    a_pack = lax.bitcast_convert_type(
        a_tab.reshape(n, d // 2, 2), jnp.float32)
    b_pack = lax.bitcast_convert_type(
        b_tab.reshape(n, d // 2, 2), jnp.float32)
    w1c_e, w1c_o = w1c[:, 0::2], w1c[:, 1::2]
    b1_e, b1_o = b1r[:, 0::2], b1r[:, 1::2]
    w2_e, w2_o = W2[0::2], W2[1::2]
    zeros_nd = jnp.zeros((n, d), jnp.float32)

    nch0 = 62
    e0 = nch0 * nw * chunk
    rows0 = e0 // chunk

    def gather_half(lo_rows, nch_h):
        return _sc_gather(x, a_pack, b_pack,
                          idxr2[lo_rows:lo_rows + nch_h * nw],
                          idxc2[lo_rows:lo_rows + nch_h * nw],
                          chunk, nch_h)

    def dense_half(g, lo):
        e_h = g[0].shape[0]
        return _edge_dense(g[0], g[1], g[2], g[3],
                           lax.dynamic_slice_in_dim(distances, lo, e_h),
                           lax.dynamic_slice_in_dim(edge_mask, lo, e_h),
                           w1c_e, w1c_o, b1_e, b1_o, w2_e, w2_o, b2r,
                           blk=2560)

    g0 = gather_half(0, nch0)
    g1 = gather_half(rows0, n_chunks_per_w - nch0)
    y0 = dense_half(g0, 0)
    y1 = dense_half(g1, e0)
    agg_p0 = _sc_scatter(y0, idxr2[:rows0], zeros_nd, n, chunk, nch0)
    agg_p1 = _sc_scatter(y1, idxr2[rows0:], zeros_nd, n, chunk,
                         n_chunks_per_w - nch0)

    return _node_post(agg_p0[0] + agg_p1[0], agg_p0[1] + agg_p1[1], x,
                      Wn1, bn1r, Wn2, bn2r)


# speedup vs baseline: 1.3185x; 1.3185x over previous
import functools

import jax
import jax.numpy as jnp
from jax import lax
from jax.experimental import pallas as pl
from jax.experimental.pallas import tpu as pltpu
from jax.experimental.pallas import tpu_sc as plsc

MIN_NORM = 1e-15
CLIP = 1e-7


def _artanh(z):
    return 0.5 * jnp.log((1.0 + z) / (1.0 - z))

def _k1_body(x_ref, w1a_ref, w1b_ref, a_ref, b_ref):
    xb = x_ref[...]
    t = jnp.sum(xb * xb, axis=1, keepdims=True)
    pn = jnp.maximum(jnp.sqrt(t), MIN_NORM)
    z = jnp.clip(pn, -1 + CLIP, 1 - CLIP)
    s = _artanh(z) / pn
    xt = xb * s
    a_ref[...] = jnp.dot(xt, w1a_ref[...], preferred_element_type=jnp.float32)
    b_ref[...] = jnp.dot(xt, w1b_ref[...], preferred_element_type=jnp.float32)


def _node_pre(x, w1a, w1b, blk=1000):
    n, d = x.shape
    grid = n // blk
    return pl.pallas_call(
        _k1_body,
        grid=(grid,),
        in_specs=[
            pl.BlockSpec((blk, d), lambda i: (i, 0)),
            pl.BlockSpec((d, d), lambda i: (0, 0)),
            pl.BlockSpec((d, d), lambda i: (0, 0)),
        ],
        out_specs=[
            pl.BlockSpec((blk, d), lambda i: (i, 0)),
            pl.BlockSpec((blk, d), lambda i: (i, 0)),
        ],
        out_shape=[
            jax.ShapeDtypeStruct((n, d), jnp.float32),
            jax.ShapeDtypeStruct((n, d), jnp.float32),
        ],
    )(x, w1a, w1b)


def _sc_gather(x, a_tab, b_tab, idxr2, idxc2, chunk, n_chunks_per_w):
    n, d = x.shape
    nrows, ck = idxr2.shape
    e = nrows * ck
    nch = n_chunks_per_w
    mesh = plsc.VectorSubcoreMesh(core_axis_name="c", subcore_axis_name="s")
    nc, ns = mesh.num_cores, mesh.num_subcores
    nw = nc * ns
    e_per_w = e // nw

    slot_bufs = []
    for _ in range(3):
        slot_bufs += [
            pltpu.VMEM((ck,), jnp.int32),
            pltpu.VMEM((ck,), jnp.int32),
            pltpu.VMEM((ck, d), jnp.float32),
            pltpu.VMEM((ck, d), jnp.float32),
            pltpu.VMEM((ck, d), jnp.float32),
            pltpu.VMEM((ck, d), jnp.float32),
        ]
    sems = [pltpu.SemaphoreType.DMA] * 9

    @functools.partial(
        pl.kernel,
        out_type=[
            jax.ShapeDtypeStruct((e, d), jnp.float32),
            jax.ShapeDtypeStruct((e, d), jnp.float32),
            jax.ShapeDtypeStruct((e, d), jnp.float32),
        ],
        mesh=mesh,
        scratch_types=slot_bufs + sems,
    )
    def k2(x_hbm, a_hbm, b_hbm, idxr_hbm, idxc_hbm,
           xr_out, xc_out, q_out, *scr):
        bufs = [scr[6 * s:6 * s + 6] for s in range(3)]
        semi = scr[18:21]
        semg = scr[21:24]
        semw = scr[24:27]
        cid = lax.axis_index("c")
        sid = lax.axis_index("s")
        wid = sid * nc + cid

        def issue_idx(s, j):
            crow = wid * nch + j
            pltpu.async_copy(idxr_hbm.at[crow], bufs[s][0], semi[s])
            pltpu.async_copy(idxc_hbm.at[crow], bufs[s][1], semi[s])

        def wait_idx(s):
            pltpu.make_async_copy(idxr_hbm.at[0], bufs[s][0], semi[s]).wait()
            pltpu.make_async_copy(idxr_hbm.at[0], bufs[s][1], semi[s]).wait()

        def issue_gathers(s):
            ir, ic, xr_v, xc_v, aq_v, bc_v = bufs[s]
            pltpu.async_copy(x_hbm.at[ir], xr_v, semg[s])
            pltpu.async_copy(x_hbm.at[ic], xc_v, semg[s])
            pltpu.async_copy(a_hbm.at[ir], aq_v, semg[s])
            pltpu.async_copy(b_hbm.at[ic], bc_v, semg[s])

        def wait_gathers(s):
            for b in (2, 3):
                pltpu.make_async_copy(
                    xr_out.at[pl.ds(0, ck)], bufs[s][b], semg[s]).wait()
            for b in (4, 5):
                pltpu.make_async_copy(
                    xr_out.at[pl.ds(0, ck)], bufs[s][b], semg[s]).wait()

        def compute_q(s):
            aq_v, bc_v = bufs[s][4], bufs[s][5]

            def addrow(r, carry):
                for kk in range(d // 16):
                    sl = pl.ds(kk * 16, 16)
                    aq_v[r, sl] = aq_v[r, sl] + bc_v[r, sl]
                return carry

            lax.fori_loop(0, ck, addrow, 0)

        def issue_writes(s, j):
            gbase = wid * e_per_w + j * chunk
            pltpu.async_copy(bufs[s][2], xr_out.at[pl.ds(gbase, ck)], semw[s])
            pltpu.async_copy(bufs[s][3], xc_out.at[pl.ds(gbase, ck)], semw[s])
            pltpu.async_copy(bufs[s][4], q_out.at[pl.ds(gbase, ck)], semw[s])

        def wait_writes(s):
            for b in (2, 3):
                pltpu.make_async_copy(
                    bufs[s][b], xr_out.at[pl.ds(0, ck)], semw[s]).wait()
            pltpu.make_async_copy(
                bufs[s][4], q_out.at[pl.ds(0, ck)], semw[s]).wait()

        def step_full(j, sj):
            s2 = (sj + 1) % 3
            s3 = (sj + 2) % 3
            wait_gathers(sj)
            compute_q(sj)
            issue_writes(sj, j)
            wait_idx(s2)
            issue_gathers(s2)
            wait_writes(s3)
            issue_idx(s3, j + 2)

        issue_idx(0, 0)
        wait_idx(0)
        issue_gathers(0)
        issue_idx(1, 1)
        wait_gathers(0)
        compute_q(0)
        issue_writes(0, 0)
        wait_idx(1)
        issue_gathers(1)
        issue_idx(2, 2)

        assert nch >= 5
        nit = (nch - 5) // 3

        def body(jj, carry):
            j0 = 1 + 3 * jj
            step_full(j0, 1)
            step_full(j0 + 1, 2)
            step_full(j0 + 2, 0)
            return carry

        lax.fori_loop(0, nit, body, 0)
        for j in range(1 + 3 * nit, nch - 2):
            step_full(j, j % 3)
        j = nch - 2
        sj = j % 3
        wait_gathers(sj)
        compute_q(sj)
        issue_writes(sj, j)
        wait_idx((sj + 1) % 3)
        issue_gathers((sj + 1) % 3)
        j = nch - 1
        sj = j % 3
        wait_gathers(sj)
        compute_q(sj)
        issue_writes(sj, j)
        for s in range(3):
            wait_writes(s)

    return k2(x, a_tab, b_tab, idxr2, idxc2)


def _k3_body(xr_ref, xc_ref, q_ref, d_ref, em_ref,
             w1c_ref, b1_ref, w2_ref, b2_ref, y_ref):
    xr = xr_ref[...]
    xc = xc_ref[...]
    q = q_ref[...] + d_ref[...] * w1c_ref[...] + b1_ref[...]
    h = q * jax.nn.sigmoid(q)
    att = jax.nn.sigmoid(
        jnp.dot(h, w2_ref[...], preferred_element_type=jnp.float32)
        + b2_ref[...]) * em_ref[...]
    tr = jnp.sum(xr * xr, axis=1, keepdims=True)
    tc = jnp.sum(xc * xc, axis=1, keepdims=True)
    xy = jnp.sum(xr * xc, axis=1, keepdims=True)
    c1 = 1.0 + 2.0 * (-xy) + tc
    c2 = 1.0 - tr
    den = jnp.maximum(1.0 + 2.0 * (-xy) + tr * tc, MIN_NORM)
    sub = (c1 * (-xr) + c2 * xc) / den
    u = jnp.sum(sub * sub, axis=1, keepdims=True)
    sn = jnp.maximum(jnp.sqrt(u), MIN_NORM)
    z = jnp.clip(sn, -1 + CLIP, 1 - CLIP)
    fac = jnp.maximum(1.0 - tr, MIN_NORM) * _artanh(z) / sn
    y_ref[...] = (att * fac) * sub


def _edge_dense(xr, xc, q0, dist, em, w1c, b1r, w2, b2r, blk=2560):
    e, d = xr.shape
    grid = e // blk
    em_spec = pl.BlockSpec((blk, 1), lambda i: (i, 0))
    row_spec = pl.BlockSpec((blk, d), lambda i: (i, 0))
    return pl.pallas_call(
        _k3_body,
        grid=(grid,),
        in_specs=[
            row_spec, row_spec, row_spec,
            em_spec, em_spec,
            pl.BlockSpec((1, d), lambda i: (0, 0)),
            pl.BlockSpec((1, d), lambda i: (0, 0)),
            pl.BlockSpec((d, 1), lambda i: (0, 0)),
            pl.BlockSpec((1, 1), lambda i: (0, 0)),
        ],
        out_specs=row_spec,
        out_shape=jax.ShapeDtypeStruct((e, d), jnp.float32),
    )(xr, xc, q0, dist, em, w1c, b1r, w2, b2r)


def _sc_scatter(y, idxr2, zeros_nd, n, chunk, n_chunks_per_w):
    e, d = y.shape
    nch = n_chunks_per_w
    mesh = plsc.VectorSubcoreMesh(core_axis_name="c", subcore_axis_name="s")
    nc, ns = mesh.num_cores, mesh.num_subcores
    nw = nc * ns
    e_per_w = e // nw
    bs = (n // ns) & ~7
    tail = n - ns * bs

    slot_bufs = []
    for _ in range(3):
        slot_bufs += [
            pltpu.VMEM((chunk,), jnp.int32),
            pltpu.VMEM((chunk, d), jnp.float32),
        ]

    @functools.partial(
        pl.kernel,
        out_type=jax.ShapeDtypeStruct((nc, n, d), jnp.float32),
        mesh=mesh,
        scratch_types=slot_bufs + [pltpu.VMEM_SHARED((n, d), jnp.float32)]
        + [pltpu.SemaphoreType.DMA] * 6,
    )
    def k4(y_hbm, idxr_hbm, z_hbm, out_hbm, *scr):
        bufs = [scr[2 * s:2 * s + 2] for s in range(3)]
        agg_sh = scr[6]
        seml = scr[7:10]
        semsc = scr[10:13]
        cid = lax.axis_index("c")
        sid = lax.axis_index("s")
        wid = sid * nc + cid
        myrows = pl.ds(sid * bs, bs)
        tailrows = pl.ds(ns * bs, tail)
        pltpu.sync_copy(z_hbm.at[myrows], agg_sh.at[myrows])
        if tail:
            @pl.when(sid == ns - 1)
            def _():
                pltpu.sync_copy(z_hbm.at[tailrows], agg_sh.at[tailrows])
        plsc.subcore_barrier()

        def issue_loads(s, j):
            crow = wid * nch + j
            gbase = wid * e_per_w + j * chunk
            pltpu.async_copy(idxr_hbm.at[crow], bufs[s][0], seml[s])
            pltpu.async_copy(y_hbm.at[pl.ds(gbase, chunk)], bufs[s][1],
                             seml[s])

        def wait_loads(s):
            pltpu.make_async_copy(idxr_hbm.at[0], bufs[s][0], seml[s]).wait()
            pltpu.make_async_copy(y_hbm.at[pl.ds(0, chunk)], bufs[s][1],
                                  seml[s]).wait()

        def issue_scatter(s):
            pltpu.async_copy(bufs[s][1], agg_sh.at[bufs[s][0]], semsc[s],
                             add=True)

        def wait_scatter(s):
            pltpu.make_async_copy(bufs[s][1], agg_sh.at[bufs[s][0]],
                                  semsc[s]).wait()

        def step_full(j, sj):
            s3 = (sj + 2) % 3
            wait_loads(sj)
            issue_scatter(sj)
            wait_scatter(s3)
            issue_loads(s3, j + 2)

        issue_loads(0, 0)
        issue_loads(1, 1)
        wait_loads(0)
        issue_scatter(0)
        issue_loads(2, 2)

        assert nch >= 5
        nit = (nch - 5) // 3

        def body(jj, carry):
            j0 = 1 + 3 * jj
            step_full(j0, 1)
            step_full(j0 + 1, 2)
            step_full(j0 + 2, 0)
            return carry

        lax.fori_loop(0, nit, body, 0)
        for j in range(1 + 3 * nit, nch - 2):
            step_full(j, j % 3)
        j = nch - 2
        wait_loads(j % 3)
        issue_scatter(j % 3)
        j = nch - 1
        wait_loads(j % 3)
        issue_scatter(j % 3)
        for s in range(3):
            wait_scatter(s)
        plsc.subcore_barrier()
        pltpu.sync_copy(agg_sh.at[myrows], out_hbm.at[cid, myrows])
        if tail:
            @pl.when(sid == ns - 1)
            def _():
                pltpu.sync_copy(agg_sh.at[tailrows], out_hbm.at[cid, tailrows])

    return k4(y, idxr2, zeros_nd)


def _k5_body(a0_ref, a1_ref, x_ref, wn1_ref, bn1_ref, wn2_ref, bn2_ref,
             out_ref):
    agg = (a0_ref[...] + a1_ref[...]) * 0.01
    m1 = jnp.dot(agg, wn1_ref[...], preferred_element_type=jnp.float32) \
        + bn1_ref[...]
    m1 = m1 * jax.nn.sigmoid(m1)
    m = jnp.dot(m1, wn2_ref[...], preferred_element_type=jnp.float32) \
        + bn2_ref[...]
    xb = x_ref[...]
    t = jnp.sum(xb * xb, axis=1, keepdims=True)
    n1 = 1.0 + 2.0 * (-t) + t
    n2 = 1.0 - t
    num = n1 * (-xb) + n2 * xb
    den = jnp.maximum(1.0 + 2.0 * (-t) + t * t, MIN_NORM)
    sb = num / den
    sn = jnp.maximum(jnp.sqrt(jnp.sum(sb * sb, axis=1, keepdims=True)),
                     MIN_NORM)
    lam = 2.0 / jnp.maximum(1.0 - t, MIN_NORM)
    zc = jnp.clip(sn, -1 + CLIP, 1 - CLIP)
    xlst = (2.0 / lam) * _artanh(zc) * sb / sn
    sup = xlst + m
    un = jnp.maximum(jnp.sqrt(jnp.sum(sup * sup, axis=1, keepdims=True)),
                     MIN_NORM)
    second = jnp.tanh(lam * un / 2.0) * sup / un
    y2 = jnp.sum(second * second, axis=1, keepdims=True)
    xy = jnp.sum(xb * second, axis=1, keepdims=True)
    numo = (1.0 + 2.0 * xy + y2) * xb + (1.0 - t) * second
    deno = jnp.maximum(1.0 + 2.0 * xy + t * y2, MIN_NORM)
    out_ref[...] = numo / deno


def _node_post(a0, a1, x, wn1, bn1, wn2, bn2, blk=1000):
    n, d = x.shape
    grid = n // blk
    row_spec = pl.BlockSpec((blk, d), lambda i: (i, 0))
    w_spec = pl.BlockSpec((d, d), lambda i: (0, 0))
    b_spec = pl.BlockSpec((1, d), lambda i: (0, 0))
    return pl.pallas_call(
        _k5_body,
        grid=(grid,),
        in_specs=[row_spec, row_spec, row_spec, w_spec, b_spec, w_spec,
                  b_spec],
        out_specs=row_spec,
        out_shape=jax.ShapeDtypeStruct((n, d), jnp.float32),
    )(a0, a1, x, wn1, bn1, wn2, bn2)


def kernel(x, distances, edges, node_mask, edge_mask, W1, b1, W2, b2,
           Wn1, bn1, Wn2, bn2):
    n, d = x.shape
    e = distances.shape[0]
    chunk = 80
    nw = 32
    e_per_w = e // nw
    n_chunks_per_w = e_per_w // chunk

    row = edges[0].astype(jnp.int32)
    col = edges[1].astype(jnp.int32)
    idxr2 = row.reshape(e // chunk, chunk)
    idxc2 = col.reshape(e // chunk, chunk)

    w1a = W1[:d]
    w1b = W1[d:2 * d]
    w1c = W1[2 * d:].reshape(1, d)
    b1r = b1.reshape(1, d)
    b2r = b2.reshape(1, 1)
    bn1r = bn1.reshape(1, d)
    bn2r = bn2.reshape(1, d)

    a_tab, b_tab = _node_pre(x, w1a, w1b)
    zeros_nd = jnp.zeros((n, d), jnp.float32)

    nch0 = 62
    e0 = nch0 * nw * chunk
    rows0 = e0 // chunk

    def gather_half(lo_rows, nch_h):
        return _sc_gather(x, a_tab, b_tab,
                          idxr2[lo_rows:lo_rows + nch_h * nw],
                          idxc2[lo_rows:lo_rows + nch_h * nw],
                          chunk, nch_h)

    def dense_half(g, lo):
        e_h = g[0].shape[0]
        return _edge_dense(g[0], g[1], g[2],
                           lax.dynamic_slice_in_dim(distances, lo, e_h),
                           lax.dynamic_slice_in_dim(edge_mask, lo, e_h),
                           w1c, b1r, W2, b2r, blk=2560)

    g0 = gather_half(0, nch0)
    g1 = gather_half(rows0, n_chunks_per_w - nch0)
    y0 = dense_half(g0, 0)
    y1 = dense_half(g1, e0)
    agg_p0 = _sc_scatter(y0, idxr2[:rows0], zeros_nd, n, chunk, nch0)
    agg_p1 = _sc_scatter(y1, idxr2[rows0:], zeros_nd, n, chunk,
                         n_chunks_per_w - nch0)

    return _node_post(agg_p0[0] + agg_p1[0], agg_p0[1] + agg_p1[1], x,
                      Wn1, bn1r, Wn2, bn2r)


# speedup vs baseline: 1.3201x; 1.0013x over previous
import functools

import jax
import jax.numpy as jnp
from jax import lax
from jax.experimental import pallas as pl
from jax.experimental.pallas import tpu as pltpu
from jax.experimental.pallas import tpu_sc as plsc

MIN_NORM = 1e-15
CLIP = 1e-7


def _artanh(z):
    return 0.5 * jnp.log((1.0 + z) / (1.0 - z))

def _k1_body(x_ref, w1a_ref, w1b_ref, a_ref, b_ref):
    xb = x_ref[...]
    t = jnp.sum(xb * xb, axis=1, keepdims=True)
    pn = jnp.maximum(jnp.sqrt(t), MIN_NORM)
    z = jnp.clip(pn, -1 + CLIP, 1 - CLIP)
    s = _artanh(z) / pn
    xt = xb * s
    a_ref[...] = jnp.dot(xt, w1a_ref[...], preferred_element_type=jnp.float32)
    b_ref[...] = jnp.dot(xt, w1b_ref[...], preferred_element_type=jnp.float32)


def _node_pre(x, w1a, w1b, blk=1000):
    n, d = x.shape
    grid = n // blk
    return pl.pallas_call(
        _k1_body,
        grid=(grid,),
        in_specs=[
            pl.BlockSpec((blk, d), lambda i: (i, 0)),
            pl.BlockSpec((d, d), lambda i: (0, 0)),
            pl.BlockSpec((d, d), lambda i: (0, 0)),
        ],
        out_specs=[
            pl.BlockSpec((blk, d), lambda i: (i, 0)),
            pl.BlockSpec((blk, d), lambda i: (i, 0)),
        ],
        out_shape=[
            jax.ShapeDtypeStruct((n, d), jnp.float32),
            jax.ShapeDtypeStruct((n, d), jnp.float32),
        ],
    )(x, w1a, w1b)


def _sc_gather(x, a_tab, b_tab, idxr2, idxc2, chunk, n_chunks_per_w):
    n, d = x.shape
    nrows, ck = idxr2.shape
    e = nrows * ck
    nch = n_chunks_per_w
    mesh = plsc.VectorSubcoreMesh(core_axis_name="c", subcore_axis_name="s")
    nc, ns = mesh.num_cores, mesh.num_subcores
    nw = nc * ns
    e_per_w = e // nw

    slot_bufs = []
    for _ in range(3):
        slot_bufs += [
            pltpu.VMEM((ck,), jnp.int32),
            pltpu.VMEM((ck,), jnp.int32),
            pltpu.VMEM((ck, d), jnp.float32),
            pltpu.VMEM((ck, d), jnp.float32),
            pltpu.VMEM((ck, d), jnp.float32),
            pltpu.VMEM((ck, d), jnp.float32),
        ]
    sems = [pltpu.SemaphoreType.DMA] * 9

    @functools.partial(
        pl.kernel,
        out_type=[
            jax.ShapeDtypeStruct((e, d), jnp.float32),
            jax.ShapeDtypeStruct((e, d), jnp.float32),
            jax.ShapeDtypeStruct((e, d), jnp.float32),
        ],
        mesh=mesh,
        scratch_types=slot_bufs + sems,
    )
    def k2(x_hbm, a_hbm, b_hbm, idxr_hbm, idxc_hbm,
           xr_out, xc_out, q_out, *scr):
        bufs = [scr[6 * s:6 * s + 6] for s in range(3)]
        semi = scr[18:21]
        semg = scr[21:24]
        semw = scr[24:27]
        cid = lax.axis_index("c")
        sid = lax.axis_index("s")
        wid = sid * nc + cid

        def issue_idx(s, j):
            crow = wid * nch + j
            pltpu.async_copy(idxr_hbm.at[crow], bufs[s][0], semi[s])
            pltpu.async_copy(idxc_hbm.at[crow], bufs[s][1], semi[s])

        def wait_idx(s):
            pltpu.make_async_copy(idxr_hbm.at[0], bufs[s][0], semi[s]).wait()
            pltpu.make_async_copy(idxr_hbm.at[0], bufs[s][1], semi[s]).wait()

        def issue_gathers(s):
            ir, ic, xr_v, xc_v, aq_v, bc_v = bufs[s]
            pltpu.async_copy(x_hbm.at[ir], xr_v, semg[s])
            pltpu.async_copy(x_hbm.at[ic], xc_v, semg[s])
            pltpu.async_copy(a_hbm.at[ir], aq_v, semg[s])
            pltpu.async_copy(b_hbm.at[ic], bc_v, semg[s])

        def wait_gathers(s):
            for b in (2, 3):
                pltpu.make_async_copy(
                    xr_out.at[pl.ds(0, ck)], bufs[s][b], semg[s]).wait()
            for b in (4, 5):
                pltpu.make_async_copy(
                    xr_out.at[pl.ds(0, ck)], bufs[s][b], semg[s]).wait()

        def compute_q(s):
            aq_v, bc_v = bufs[s][4], bufs[s][5]

            def addrow(r, carry):
                for kk in range(d // 16):
                    sl = pl.ds(kk * 16, 16)
                    aq_v[r, sl] = aq_v[r, sl] + bc_v[r, sl]
                return carry

            lax.fori_loop(0, ck, addrow, 0)

        def issue_writes(s, j):
            gbase = wid * e_per_w + j * chunk
            pltpu.async_copy(bufs[s][2], xr_out.at[pl.ds(gbase, ck)], semw[s])
            pltpu.async_copy(bufs[s][3], xc_out.at[pl.ds(gbase, ck)], semw[s])
            pltpu.async_copy(bufs[s][4], q_out.at[pl.ds(gbase, ck)], semw[s])

        def wait_writes(s):
            for b in (2, 3):
                pltpu.make_async_copy(
                    bufs[s][b], xr_out.at[pl.ds(0, ck)], semw[s]).wait()
            pltpu.make_async_copy(
                bufs[s][4], q_out.at[pl.ds(0, ck)], semw[s]).wait()

        def step_full(j, sj):
            s2 = (sj + 1) % 3
            s3 = (sj + 2) % 3
            wait_gathers(sj)
            compute_q(sj)
            issue_writes(sj, j)
            wait_idx(s2)
            issue_gathers(s2)
            wait_writes(s3)
            issue_idx(s3, j + 2)

        issue_idx(0, 0)
        wait_idx(0)
        issue_gathers(0)
        issue_idx(1, 1)
        wait_gathers(0)
        compute_q(0)
        issue_writes(0, 0)
        wait_idx(1)
        issue_gathers(1)
        issue_idx(2, 2)

        assert nch >= 5
        nit = (nch - 5) // 3

        def body(jj, carry):
            j0 = 1 + 3 * jj
            step_full(j0, 1)
            step_full(j0 + 1, 2)
            step_full(j0 + 2, 0)
            return carry

        lax.fori_loop(0, nit, body, 0)
        for j in range(1 + 3 * nit, nch - 2):
            step_full(j, j % 3)
        j = nch - 2
        sj = j % 3
        wait_gathers(sj)
        compute_q(sj)
        issue_writes(sj, j)
        wait_idx((sj + 1) % 3)
        issue_gathers((sj + 1) % 3)
        j = nch - 1
        sj = j % 3
        wait_gathers(sj)
        compute_q(sj)
        issue_writes(sj, j)
        for s in range(3):
            wait_writes(s)

    return k2(x, a_tab, b_tab, idxr2, idxc2)


def _k3_body(xr_ref, xc_ref, q_ref, d_ref, em_ref,
             w1c_ref, b1_ref, w2_ref, b2_ref, y_ref):
    xr = xr_ref[...]
    xc = xc_ref[...]
    q = q_ref[...] + d_ref[...] * w1c_ref[...] + b1_ref[...]
    h = q * jax.nn.sigmoid(q)
    att = jax.nn.sigmoid(
        jnp.dot(h, w2_ref[...], preferred_element_type=jnp.float32)
        + b2_ref[...]) * em_ref[...]
    tr = jnp.sum(xr * xr, axis=1, keepdims=True)
    tc = jnp.sum(xc * xc, axis=1, keepdims=True)
    xy = jnp.sum(xr * xc, axis=1, keepdims=True)
    c1 = 1.0 + 2.0 * (-xy) + tc
    c2 = 1.0 - tr
    den = jnp.maximum(1.0 + 2.0 * (-xy) + tr * tc, MIN_NORM)
    sub = (c1 * (-xr) + c2 * xc) / den
    u = jnp.sum(sub * sub, axis=1, keepdims=True)
    sn = jnp.maximum(jnp.sqrt(u), MIN_NORM)
    z = jnp.clip(sn, -1 + CLIP, 1 - CLIP)
    fac = jnp.maximum(1.0 - tr, MIN_NORM) * _artanh(z) / sn
    y_ref[...] = (att * fac) * sub


def _edge_dense(xr, xc, q0, dist, em, w1c, b1r, w2, b2r, blk=2560):
    e, d = xr.shape
    grid = e // blk
    em_spec = pl.BlockSpec((blk, 1), lambda i: (i, 0))
    row_spec = pl.BlockSpec((blk, d), lambda i: (i, 0))
    return pl.pallas_call(
        _k3_body,
        grid=(grid,),
        in_specs=[
            row_spec, row_spec, row_spec,
            em_spec, em_spec,
            pl.BlockSpec((1, d), lambda i: (0, 0)),
            pl.BlockSpec((1, d), lambda i: (0, 0)),
            pl.BlockSpec((d, 1), lambda i: (0, 0)),
            pl.BlockSpec((1, 1), lambda i: (0, 0)),
        ],
        out_specs=row_spec,
        out_shape=jax.ShapeDtypeStruct((e, d), jnp.float32),
    )(xr, xc, q0, dist, em, w1c, b1r, w2, b2r)


def _sc_scatter(y, idxr2, zeros_nd, n, chunk, n_chunks_per_w):
    e, d = y.shape
    nch = n_chunks_per_w
    mesh = plsc.VectorSubcoreMesh(core_axis_name="c", subcore_axis_name="s")
    nc, ns = mesh.num_cores, mesh.num_subcores
    nw = nc * ns
    e_per_w = e // nw
    bs = (n // ns) & ~7
    tail = n - ns * bs

    slot_bufs = []
    for _ in range(3):
        slot_bufs += [
            pltpu.VMEM((chunk,), jnp.int32),
            pltpu.VMEM((chunk, d), jnp.float32),
        ]

    @functools.partial(
        pl.kernel,
        out_type=jax.ShapeDtypeStruct((nc, n, d), jnp.float32),
        mesh=mesh,
        scratch_types=slot_bufs + [pltpu.VMEM_SHARED((n, d), jnp.float32)]
        + [pltpu.SemaphoreType.DMA] * 6,
    )
    def k4(y_hbm, idxr_hbm, z_hbm, out_hbm, *scr):
        bufs = [scr[2 * s:2 * s + 2] for s in range(3)]
        agg_sh = scr[6]
        seml = scr[7:10]
        semsc = scr[10:13]
        cid = lax.axis_index("c")
        sid = lax.axis_index("s")
        wid = sid * nc + cid
        myrows = pl.ds(sid * bs, bs)
        tailrows = pl.ds(ns * bs, tail)
        pltpu.sync_copy(z_hbm.at[myrows], agg_sh.at[myrows])
        if tail:
            @pl.when(sid == ns - 1)
            def _():
                pltpu.sync_copy(z_hbm.at[tailrows], agg_sh.at[tailrows])
        plsc.subcore_barrier()

        def issue_loads(s, j):
            crow = wid * nch + j
            gbase = wid * e_per_w + j * chunk
            pltpu.async_copy(idxr_hbm.at[crow], bufs[s][0], seml[s])
            pltpu.async_copy(y_hbm.at[pl.ds(gbase, chunk)], bufs[s][1],
                             seml[s])

        def wait_loads(s):
            pltpu.make_async_copy(idxr_hbm.at[0], bufs[s][0], seml[s]).wait()
            pltpu.make_async_copy(y_hbm.at[pl.ds(0, chunk)], bufs[s][1],
                                  seml[s]).wait()

        def issue_scatter(s):
            pltpu.async_copy(bufs[s][1], agg_sh.at[bufs[s][0]], semsc[s],
                             add=True)

        def wait_scatter(s):
            pltpu.make_async_copy(bufs[s][1], agg_sh.at[bufs[s][0]],
                                  semsc[s]).wait()

        def step_full(j, sj):
            s3 = (sj + 2) % 3
            wait_loads(sj)
            issue_scatter(sj)
            wait_scatter(s3)
            issue_loads(s3, j + 2)

        issue_loads(0, 0)
        issue_loads(1, 1)
        wait_loads(0)
        issue_scatter(0)
        issue_loads(2, 2)

        assert nch >= 5
        nit = (nch - 5) // 3

        def body(jj, carry):
            j0 = 1 + 3 * jj
            step_full(j0, 1)
            step_full(j0 + 1, 2)
            step_full(j0 + 2, 0)
            return carry

        lax.fori_loop(0, nit, body, 0)
        for j in range(1 + 3 * nit, nch - 2):
            step_full(j, j % 3)
        j = nch - 2
        wait_loads(j % 3)
        issue_scatter(j % 3)
        j = nch - 1
        wait_loads(j % 3)
        issue_scatter(j % 3)
        for s in range(3):
            wait_scatter(s)
        plsc.subcore_barrier()
        pltpu.sync_copy(agg_sh.at[myrows], out_hbm.at[cid, myrows])
        if tail:
            @pl.when(sid == ns - 1)
            def _():
                pltpu.sync_copy(agg_sh.at[tailrows], out_hbm.at[cid, tailrows])

    return k4(y, idxr2, zeros_nd)


def _k5_body(a0_ref, a1_ref, x_ref, wn1_ref, bn1_ref, wn2_ref, bn2_ref,
             out_ref):
    agg = (a0_ref[...] + a1_ref[...]) * 0.01
    m1 = jnp.dot(agg, wn1_ref[...], preferred_element_type=jnp.float32) \
        + bn1_ref[...]
    m1 = m1 * jax.nn.sigmoid(m1)
    m = jnp.dot(m1, wn2_ref[...], preferred_element_type=jnp.float32) \
        + bn2_ref[...]
    xb = x_ref[...]
    t = jnp.sum(xb * xb, axis=1, keepdims=True)
    n1 = 1.0 + 2.0 * (-t) + t
    n2 = 1.0 - t
    num = n1 * (-xb) + n2 * xb
    den = jnp.maximum(1.0 + 2.0 * (-t) + t * t, MIN_NORM)
    sb = num / den
    sn = jnp.maximum(jnp.sqrt(jnp.sum(sb * sb, axis=1, keepdims=True)),
                     MIN_NORM)
    lam = 2.0 / jnp.maximum(1.0 - t, MIN_NORM)
    zc = jnp.clip(sn, -1 + CLIP, 1 - CLIP)
    xlst = (2.0 / lam) * _artanh(zc) * sb / sn
    sup = xlst + m
    un = jnp.maximum(jnp.sqrt(jnp.sum(sup * sup, axis=1, keepdims=True)),
                     MIN_NORM)
    second = jnp.tanh(lam * un / 2.0) * sup / un
    y2 = jnp.sum(second * second, axis=1, keepdims=True)
    xy = jnp.sum(xb * second, axis=1, keepdims=True)
    numo = (1.0 + 2.0 * xy + y2) * xb + (1.0 - t) * second
    deno = jnp.maximum(1.0 + 2.0 * xy + t * y2, MIN_NORM)
    out_ref[...] = numo / deno


def _node_post(a0, a1, x, wn1, bn1, wn2, bn2, blk=1000):
    n, d = x.shape
    grid = n // blk
    row_spec = pl.BlockSpec((blk, d), lambda i: (i, 0))
    w_spec = pl.BlockSpec((d, d), lambda i: (0, 0))
    b_spec = pl.BlockSpec((1, d), lambda i: (0, 0))
    return pl.pallas_call(
        _k5_body,
        grid=(grid,),
        in_specs=[row_spec, row_spec, row_spec, w_spec, b_spec, w_spec,
                  b_spec],
        out_specs=row_spec,
        out_shape=jax.ShapeDtypeStruct((n, d), jnp.float32),
    )(a0, a1, x, wn1, bn1, wn2, bn2)


def kernel(x, distances, edges, node_mask, edge_mask, W1, b1, W2, b2,
           Wn1, bn1, Wn2, bn2):
    n, d = x.shape
    e = distances.shape[0]
    chunk = 80
    nw = 32
    e_per_w = e // nw
    n_chunks_per_w = e_per_w // chunk

    row = edges[0].astype(jnp.int32)
    col = edges[1].astype(jnp.int32)
    idxr2 = row.reshape(e // chunk, chunk)
    idxc2 = col.reshape(e // chunk, chunk)

    w1a = W1[:d]
    w1b = W1[d:2 * d]
    w1c = W1[2 * d:].reshape(1, d)
    b1r = b1.reshape(1, d)
    b2r = b2.reshape(1, 1)
    bn1r = bn1.reshape(1, d)
    bn2r = bn2.reshape(1, d)

    a_tab, b_tab = _node_pre(x, w1a, w1b)
    zeros_nd = jnp.zeros((n, d), jnp.float32)

    pieces = [41, 42, 42]

    def gather_piece(lo_rows, nch_h):
        return _sc_gather(x, a_tab, b_tab,
                          idxr2[lo_rows:lo_rows + nch_h * nw],
                          idxc2[lo_rows:lo_rows + nch_h * nw],
                          chunk, nch_h)

    def dense_piece(g, lo):
        e_h = g[0].shape[0]
        return _edge_dense(g[0], g[1], g[2],
                           lax.dynamic_slice_in_dim(distances, lo, e_h),
                           lax.dynamic_slice_in_dim(edge_mask, lo, e_h),
                           w1c, b1r, W2, b2r, blk=2560)

    gs, ys, aggs = [], [], []
    lo_rows = 0
    for nch_h in pieces:
        gs.append((gather_piece(lo_rows, nch_h), lo_rows, nch_h))
        lo_rows += nch_h * nw
    for g, lo_rows, nch_h in gs:
        ys.append((dense_piece(g, lo_rows * chunk), lo_rows, nch_h))
    for y, lo_rows, nch_h in ys:
        aggs.append(_sc_scatter(y, idxr2[lo_rows:lo_rows + nch_h * nw],
                                zeros_nd, n, chunk, nch_h))

    a0 = aggs[0][0] + aggs[1][0] + aggs[2][0]
    a1 = aggs[0][1] + aggs[1][1] + aggs[2][1]
    return _node_post(a0, a1, x, Wn1, bn1r, Wn2, bn2r)
